# baseline (speedup 1.0000x reference)
import functools

import jax
import jax.numpy as jnp
from jax import lax
from jax.experimental import pallas as pl
from jax.experimental.pallas import tpu as pltpu
from jax.experimental.pallas import tpu_sc as plsc

NR_CONCEPTS = 1000000
CONCEPT_DIM = 32
BATCH = 4096
HIST = 200
NB = BATCH * HIST


def kernel(x, weight):
    info = plsc.get_sparse_core_info()
    nw = info.num_cores * info.num_subcores
    b_per_w = NB // nw
    chunk = 1024
    n_chunks = b_per_w // chunk
    nbuf = 3

    mesh = plsc.VectorSubcoreMesh(core_axis_name="c", subcore_axis_name="s")

    @functools.partial(
        pl.kernel,
        mesh=mesh,
        out_type=jax.ShapeDtypeStruct((NB, CONCEPT_DIM), jnp.float32),
        scratch_types=(
            [pltpu.VMEM((b_per_w,), jnp.int32)]
            + [pltpu.VMEM((chunk, CONCEPT_DIM), jnp.float32)] * nbuf
            + [pltpu.SemaphoreType.DMA] * (2 * nbuf)
        ),
        compiler_params=pltpu.CompilerParams(use_tc_tiling_on_sc=False),
    )
    def emb_kernel(idx_hbm, table_hbm, out_hbm, idx_all, *bufs):
        rows = bufs[:nbuf]
        gsem = bufs[nbuf:2 * nbuf]
        wsem = bufs[2 * nbuf:]
        cid = lax.axis_index("c")
        sid = lax.axis_index("s")
        wid = sid * info.num_cores + cid
        base = wid * b_per_w

        pltpu.sync_copy(idx_hbm.at[pl.ds(base, b_per_w)], idx_all)

        def start_gather(i, b):
            return pltpu.async_copy(
                table_hbm.at[idx_all.at[pl.ds(i * chunk, chunk)]],
                rows[b], gsem[b])

        def start_write(i, b):
            return pltpu.async_copy(
                rows[b], out_hbm.at[pl.ds(base + i * chunk, chunk)], wsem[b])

        depth = nbuf - 1
        g = [None] * nbuf
        w = [None] * nbuf
        for j in range(depth):
            g[j % nbuf] = start_gather(j, j % nbuf)
        for i in range(n_chunks):
            b = i % nbuf
            g[b].wait()
            w[b] = start_write(i, b)
            j = i + depth
            if j < n_chunks:
                nb = j % nbuf
                if w[nb] is not None:
                    w[nb].wait()
                g[nb] = start_gather(j, nb)
        for b in range(nbuf):
            if w[b] is not None:
                w[b].wait()

    out = emb_kernel(x.reshape(NB), weight)
    return out.reshape(BATCH, HIST, CONCEPT_DIM)
